# Initial kernel scaffold; baseline (speedup 1.0000x reference)
import jax
import jax.numpy as jnp
from jax.experimental import pallas as pl


def kernel(x, edge_index, batch, params):
    raise NotImplementedError("write your pallas kernel here")


# speedup vs baseline: 3.3863x; 3.3863x over previous
import functools

import jax
import jax.numpy as jnp
from jax import lax
from jax.experimental import pallas as pl
from jax.experimental.pallas import tpu as pltpu
from jax.experimental.pallas import tpu_sc as plsc

N = 10000
E = 320000
D = 128
H = 64
G = 8
EPS = 1e-5

NC = 2
NS = 16
NW = NC * NS
CH = 128
NCHUNK = 2560
EPAD = NCHUNK * CH
CPW = NCHUNK // NW
HALF = CPW // 2
NB = 2
NP = 10240
RPS = NP // NS
W = 128


@functools.lru_cache(maxsize=None)
def _make_spmm():
    mesh = plsc.VectorSubcoreMesh(core_axis_name="c", subcore_axis_name="s",
                                  num_cores=NC, num_subcores=NS)
    out_type = jax.ShapeDtypeStruct((2 * NP, W), jnp.float32)
    scratch = [
        pltpu.VMEM((HALF, CH), jnp.int32),
        pltpu.VMEM((HALF, CH), jnp.int32),
    ]
    scratch += [pltpu.VMEM((CH, W), jnp.float32) for _ in range(NB)]
    scratch += [pltpu.SemaphoreType.DMA for _ in range(2 * NB)]
    scratch += [pltpu.VMEM_SHARED((NP, W), jnp.float32)]

    def body(hlp, src_i, dst_i, agg_o, src_v, dst_v, *rest):
        rows = rest[:NB]
        sems = rest[NB:NB + 2 * NB]
        agg_sh = rest[NB + 2 * NB]
        sg = sems[:NB]
        ss = sems[NB:]

        cid = lax.axis_index("c")
        sid = lax.axis_index("s")
        w = cid * NS + sid

        @pl.loop(0, CH)
        def _(r):
            for c in range(W // 16):
                rows[0][r, pl.ds(c * 16, 16)] = jnp.zeros((16,), jnp.float32)

        @pl.loop(0, RPS // CH)
        def _(i):
            pltpu.sync_copy(rows[0], agg_sh.at[pl.ds(sid * RPS + i * CH, CH)])

        plsc.subcore_barrier()

        for phase in range(CPW // HALF):
            base_row = w * CPW + phase * HALF
            pltpu.sync_copy(src_i.at[pl.ds(base_row, HALF)], src_v)
            pltpu.sync_copy(dst_i.at[pl.ds(base_row, HALF)], dst_v)

            for b in range(NB):
                pltpu.async_copy(hlp.at[src_v.at[b]], rows[b], sg[b])

            @pl.loop(0, HALF, step=NB)
            def _(base):
                for b in range(NB):
                    k = base + b
                    pltpu.make_async_copy(hlp.at[src_v.at[0]], rows[b],
                                          sg[b]).wait()
                    pltpu.async_copy(rows[b], agg_sh.at[dst_v.at[k]], ss[b],
                                     add=True)
                for b in range(NB):
                    nk = base + NB + b
                    pltpu.make_async_copy(rows[b], agg_sh.at[dst_v.at[0]],
                                          ss[b]).wait()

                    @pl.when(nk < HALF)
                    def _():
                        pltpu.async_copy(hlp.at[src_v.at[nk]], rows[b], sg[b])

        plsc.subcore_barrier()

        pltpu.sync_copy(agg_sh.at[pl.ds(sid * RPS, RPS)],
                        agg_o.at[pl.ds(cid * NP + sid * RPS, RPS)])

    return pl.kernel(body, out_type=out_type, mesh=mesh,
                     scratch_types=scratch)


BN = 2000
NBLK = N // BN


def _dot_t(a, w):
    return lax.dot_general(a, w, (((1,), (1,)), ((), ())),
                           preferred_element_type=jnp.float32)


def _pad_cols(hl, deg_lane):
    if deg_lane:
        extra = (lax.broadcasted_iota(jnp.int32, (BN, W - H), 1) == 0
                 ).astype(jnp.float32)
    else:
        extra = jnp.zeros((BN, W - H), jnp.float32)
    return jnp.concatenate([hl, extra], axis=1)


def _proj0_body(x_ref, wl_ref, wr_ref, bl_ref, hlp_ref, hr_ref):
    xb = x_ref[...]
    hlp_ref[...] = _pad_cols(_dot_t(xb, wl_ref[...]), True)
    hr_ref[...] = _dot_t(xb, wr_ref[...]) + bl_ref[...]


def _norm_block(a0, a1, d0, d1, hr, g, b):
    deg = jnp.maximum(d0[...][:, 0:1] + d1[...][:, 0:1], 1.0)
    t = (a0[...][:, :H] + a1[...][:, :H]) / deg + hr[...]
    m = jnp.mean(t, axis=-1, keepdims=True)
    v = jnp.mean((t - m) ** 2, axis=-1, keepdims=True)
    h = (t - m) * lax.rsqrt(v + EPS) * g[...] + b[...]
    return jnp.maximum(h, 0.0)


def _mid_body(a0, a1, d0, d1, hr, g, b, wl, wr, bln, hlp_o, hr_o):
    h = _norm_block(a0, a1, d0, d1, hr, g, b)
    hlp_o[...] = _pad_cols(_dot_t(h, wl[...]), False)
    hr_o[...] = _dot_t(h, wr[...]) + bln[...]


def _final_body(a0, a1, d0, d1, hr, g, b, batch_ref, node_o, graph_o,
                sums_acc, cnt_acc):
    i = pl.program_id(0)
    h = _norm_block(a0, a1, d0, d1, hr, g, b)
    node_o[...] = h
    bi = batch_ref[0]
    oh = (lax.broadcasted_iota(jnp.int32, (G, BN), 0) == bi
          ).astype(jnp.float32)
    s = lax.dot_general(oh, h, (((1,), (0,)), ((), ())),
                        preferred_element_type=jnp.float32)
    c = jnp.sum(oh, axis=1, keepdims=True)

    @pl.when(i == 0)
    def _():
        sums_acc[...] = jnp.zeros_like(sums_acc)
        cnt_acc[...] = jnp.zeros_like(cnt_acc)

    sums_acc[...] += s
    cnt_acc[...] += jnp.broadcast_to(c, (G, H))

    @pl.when(i == NBLK - 1)
    def _():
        graph_o[...] = sums_acc[...] / jnp.maximum(cnt_acc[...], 1.0)


def _row_spec(shape):
    return pl.BlockSpec((BN,) + shape[1:],
                        lambda i: (i,) + (0,) * (len(shape) - 1))


def _full_spec(shape):
    return pl.BlockSpec(shape, lambda i: (0,) * len(shape))


def _proj0(x, wl, wr, bl):
    return pl.pallas_call(
        _proj0_body,
        grid=(NBLK,),
        in_specs=[_row_spec((N, D)), _full_spec((H, D)), _full_spec((H, D)),
                  _full_spec((1, H))],
        out_specs=[_row_spec((N, W)), _row_spec((N, H))],
        out_shape=[jax.ShapeDtypeStruct((N, W), jnp.float32),
                   jax.ShapeDtypeStruct((N, H), jnp.float32)],
    )(x, wl, wr, bl)


def _mid(a0, a1, d0, d1, hr, g, b, wl, wr, bln):
    return pl.pallas_call(
        _mid_body,
        grid=(NBLK,),
        in_specs=[_row_spec((N, W))] * 2 + [_row_spec((N, 16))] * 2 + [
            _row_spec((N, H)), _full_spec((1, H)), _full_spec((1, H)),
            _full_spec((H, H)), _full_spec((H, H)), _full_spec((1, H))],
        out_specs=[_row_spec((N, W)), _row_spec((N, H))],
        out_shape=[jax.ShapeDtypeStruct((N, W), jnp.float32),
                   jax.ShapeDtypeStruct((N, H), jnp.float32)],
    )(a0, a1, d0, d1, hr, g, b, wl, wr, bln)


def _final(a0, a1, d0, d1, hr, g, b, batch3d):
    return pl.pallas_call(
        _final_body,
        grid=(NBLK,),
        in_specs=[_row_spec((N, W))] * 2 + [_row_spec((N, 16))] * 2 + [
            _row_spec((N, H)), _full_spec((1, H)), _full_spec((1, H)),
            pl.BlockSpec((1, 1, BN), lambda i: (i, 0, 0))],
        out_specs=[_row_spec((N, H)), _full_spec((G, H))],
        out_shape=[jax.ShapeDtypeStruct((N, H), jnp.float32),
                   jax.ShapeDtypeStruct((G, H), jnp.float32)],
        scratch_shapes=[pltpu.VMEM((G, H), jnp.float32),
                        pltpu.VMEM((G, H), jnp.float32)],
    )(a0, a1, d0, d1, hr, g, b, batch3d)


def _split(parts):
    return parts[:N], parts[NP:NP + N]


def kernel(x, edge_index, batch, params):
    pad = EPAD - E
    src2d = jnp.concatenate(
        [edge_index[0], jnp.zeros((pad,), jnp.int32)]).reshape(NCHUNK, CH)
    dst2d = jnp.concatenate(
        [edge_index[1], jnp.full((pad,), N, jnp.int32)]).reshape(NCHUNK, CH)
    batch3d = batch.reshape(NBLK, 1, BN)
    (wl1, bl1, wr1, g1, b1) = params[0]
    (wl2, bl2, wr2, g2, b2) = params[1]
    (wl3, bl3, wr3, g3, b3) = params[2]
    r = lambda v: v.reshape(1, H)
    spmm = _make_spmm()

    hlp1, hr1 = _proj0(x, wl1, wr1, r(bl1))
    agg1 = spmm(hlp1, src2d, dst2d)
    a0, a1 = _split(agg1)
    d0 = agg1[:N, H:H + 16]
    d1 = agg1[NP:NP + N, H:H + 16]
    hlp2, hr2 = _mid(a0, a1, d0, d1, hr1, r(g1), r(b1), wl2, wr2, r(bl2))
    agg2 = spmm(hlp2, src2d, dst2d)
    a0, a1 = _split(agg2)
    hlp3, hr3 = _mid(a0, a1, d0, d1, hr2, r(g2), r(b2), wl3, wr3, r(bl3))
    agg3 = spmm(hlp3, src2d, dst2d)
    a0, a1 = _split(agg3)
    node_emb, graph_emb = _final(a0, a1, d0, d1, hr3, r(g3), r(b3), batch3d)
    return (graph_emb, node_emb)


# speedup vs baseline: 9.3993x; 2.7757x over previous
import functools

import jax
import jax.numpy as jnp
from jax import lax
from jax.experimental import pallas as pl
from jax.experimental.pallas import tpu as pltpu
from jax.experimental.pallas import tpu_sc as plsc

N = 10000
E = 320000
D = 128
H = 64
G = 8
EPS = 1e-5

NC = 2
NS = 16
NW = NC * NS
CH = 128
NCHUNK = 2560
EPAD = NCHUNK * CH
CPW = NCHUNK // NW
HALF = CPW // 2
NB = 2
NP = 10240
RPS = NP // NS
W = 128


@functools.lru_cache(maxsize=None)
def _make_spmm():
    mesh = plsc.VectorSubcoreMesh(core_axis_name="c", subcore_axis_name="s",
                                  num_cores=NC, num_subcores=NS)
    out_type = jax.ShapeDtypeStruct((2 * NP, W), jnp.float32)
    scratch = [
        pltpu.VMEM((HALF, CH), jnp.int32),
        pltpu.VMEM((HALF, CH), jnp.int32),
    ]
    scratch += [pltpu.VMEM((CH, W), jnp.float32) for _ in range(NB)]
    scratch += [pltpu.SemaphoreType.DMA for _ in range(2 * NB)]
    scratch += [pltpu.VMEM_SHARED((NP, W), jnp.float32)]

    def body(hlp, src_i, dst_i, agg_o, src_v, dst_v, *rest):
        rows = rest[:NB]
        sems = rest[NB:NB + 2 * NB]
        agg_sh = rest[NB + 2 * NB]
        sg = sems[:NB]
        ss = sems[NB:]

        cid = lax.axis_index("c")
        sid = lax.axis_index("s")
        w = cid * NS + sid

        @pl.loop(0, CH)
        def _(r):
            for c in range(W // 16):
                rows[0][r, pl.ds(c * 16, 16)] = jnp.zeros((16,), jnp.float32)

        @pl.loop(0, RPS // CH)
        def _(i):
            pltpu.sync_copy(rows[0], agg_sh.at[pl.ds(sid * RPS + i * CH, CH)])

        plsc.subcore_barrier()

        for phase in range(CPW // HALF):
            base_row = w * CPW + phase * HALF
            pltpu.sync_copy(src_i.at[pl.ds(base_row, HALF)], src_v)
            pltpu.sync_copy(dst_i.at[pl.ds(base_row, HALF)], dst_v)

            for b in range(NB):
                pltpu.async_copy(hlp.at[src_v.at[b]], rows[b], sg[b])

            @pl.loop(0, HALF, step=NB)
            def _(base):
                for b in range(NB):
                    k = base + b
                    pltpu.make_async_copy(hlp.at[src_v.at[0]], rows[b],
                                          sg[b]).wait()
                    pltpu.async_copy(rows[b], agg_sh.at[dst_v.at[k]], ss[b],
                                     add=True)
                for b in range(NB):
                    nk = base + NB + b
                    pltpu.make_async_copy(rows[b], agg_sh.at[dst_v.at[0]],
                                          ss[b]).wait()

                    @pl.when(nk < HALF)
                    def _():
                        pltpu.async_copy(hlp.at[src_v.at[nk]], rows[b], sg[b])

        plsc.subcore_barrier()

        pltpu.sync_copy(agg_sh.at[pl.ds(sid * RPS, RPS)],
                        agg_o.at[pl.ds(cid * NP + sid * RPS, RPS)])

    return pl.kernel(body, out_type=out_type, mesh=mesh,
                     scratch_types=scratch)


BN = 2000
NBLK = N // BN


def _dot_t(a, w):
    return lax.dot_general(a, w, (((1,), (1,)), ((), ())),
                           preferred_element_type=jnp.float32)


def _pad_cols(hl, deg_lane):
    if deg_lane:
        extra = (lax.broadcasted_iota(jnp.int32, (BN, W - H), 1) == 0
                 ).astype(jnp.float32)
    else:
        extra = jnp.zeros((BN, W - H), jnp.float32)
    return jnp.concatenate([hl, extra], axis=1)


def _proj0_body(x_ref, wl_ref, wr_ref, bl_ref, hlp_ref, hr_ref):
    xb = x_ref[...]
    hlp_ref[...] = _pad_cols(_dot_t(xb, wl_ref[...]), True)
    hr_ref[...] = _dot_t(xb, wr_ref[...]) + bl_ref[...]


def _norm_block(a0, a1, d0, d1, hr, g, b):
    deg = jnp.maximum(d0[...][:, 0:1] + d1[...][:, 0:1], 1.0)
    t = (a0[...][:, :H] + a1[...][:, :H]) / deg + hr[...]
    m = jnp.mean(t, axis=-1, keepdims=True)
    v = jnp.mean((t - m) ** 2, axis=-1, keepdims=True)
    h = (t - m) * lax.rsqrt(v + EPS) * g[...] + b[...]
    return jnp.maximum(h, 0.0)


def _mid_body(a0, a1, d0, d1, hr, g, b, wl, wr, bln, hlp_o, hr_o):
    h = _norm_block(a0, a1, d0, d1, hr, g, b)
    hlp_o[...] = _pad_cols(_dot_t(h, wl[...]), False)
    hr_o[...] = _dot_t(h, wr[...]) + bln[...]


def _final_body(a0, a1, d0, d1, hr, g, b, batch_ref, node_o, graph_o,
                sums_acc, cnt_acc):
    i = pl.program_id(0)
    h = _norm_block(a0, a1, d0, d1, hr, g, b)
    node_o[...] = h
    bi = batch_ref[0]
    oh = (lax.broadcasted_iota(jnp.int32, (G, BN), 0) == bi
          ).astype(jnp.float32)
    s = lax.dot_general(oh, h, (((1,), (0,)), ((), ())),
                        preferred_element_type=jnp.float32)
    c = jnp.sum(oh, axis=1, keepdims=True)

    @pl.when(i == 0)
    def _():
        sums_acc[...] = jnp.zeros_like(sums_acc)
        cnt_acc[...] = jnp.zeros_like(cnt_acc)

    sums_acc[...] += s
    cnt_acc[...] += jnp.broadcast_to(c, (G, H))

    @pl.when(i == NBLK - 1)
    def _():
        graph_o[...] = sums_acc[...] / jnp.maximum(cnt_acc[...], 1.0)


def _row_spec(shape):
    return pl.BlockSpec((BN,) + shape[1:],
                        lambda i: (i,) + (0,) * (len(shape) - 1))


def _full_spec(shape):
    return pl.BlockSpec(shape, lambda i: (0,) * len(shape))


def _proj0(x, wl, wr, bl):
    return pl.pallas_call(
        _proj0_body,
        grid=(NBLK,),
        in_specs=[_row_spec((N, D)), _full_spec((H, D)), _full_spec((H, D)),
                  _full_spec((1, H))],
        out_specs=[_row_spec((N, W)), _row_spec((N, H))],
        out_shape=[jax.ShapeDtypeStruct((N, W), jnp.float32),
                   jax.ShapeDtypeStruct((N, H), jnp.float32)],
    )(x, wl, wr, bl)


def _mid(a0, a1, d0, d1, hr, g, b, wl, wr, bln):
    return pl.pallas_call(
        _mid_body,
        grid=(NBLK,),
        in_specs=[_row_spec((N, W))] * 2 + [_row_spec((N, 16))] * 2 + [
            _row_spec((N, H)), _full_spec((1, H)), _full_spec((1, H)),
            _full_spec((H, H)), _full_spec((H, H)), _full_spec((1, H))],
        out_specs=[_row_spec((N, W)), _row_spec((N, H))],
        out_shape=[jax.ShapeDtypeStruct((N, W), jnp.float32),
                   jax.ShapeDtypeStruct((N, H), jnp.float32)],
    )(a0, a1, d0, d1, hr, g, b, wl, wr, bln)


def _final(a0, a1, d0, d1, hr, g, b, batch3d):
    return pl.pallas_call(
        _final_body,
        grid=(NBLK,),
        in_specs=[_row_spec((N, W))] * 2 + [_row_spec((N, 16))] * 2 + [
            _row_spec((N, H)), _full_spec((1, H)), _full_spec((1, H)),
            pl.BlockSpec((1, 1, BN), lambda i: (i, 0, 0))],
        out_specs=[_row_spec((N, H)), _full_spec((G, H))],
        out_shape=[jax.ShapeDtypeStruct((N, H), jnp.float32),
                   jax.ShapeDtypeStruct((G, H), jnp.float32)],
        scratch_shapes=[pltpu.VMEM((G, H), jnp.float32),
                        pltpu.VMEM((G, H), jnp.float32)],
    )(a0, a1, d0, d1, hr, g, b, batch3d)


def _split(parts):
    return parts[:N], parts[NP:NP + N]


def kernel(x, edge_index, batch, params):
    pad = EPAD - E
    lanes = jnp.arange(pad, dtype=jnp.int32) % CH
    src2d = jnp.concatenate(
        [edge_index[0], lanes]).reshape(NCHUNK, CH)
    dst2d = jnp.concatenate(
        [edge_index[1], N + lanes]).reshape(NCHUNK, CH)
    batch3d = batch.reshape(NBLK, 1, BN)
    (wl1, bl1, wr1, g1, b1) = params[0]
    (wl2, bl2, wr2, g2, b2) = params[1]
    (wl3, bl3, wr3, g3, b3) = params[2]
    r = lambda v: v.reshape(1, H)
    spmm = _make_spmm()

    hlp1, hr1 = _proj0(x, wl1, wr1, r(bl1))
    agg1 = spmm(hlp1, src2d, dst2d)
    a0, a1 = _split(agg1)
    d0 = agg1[:N, H:H + 16]
    d1 = agg1[NP:NP + N, H:H + 16]
    hlp2, hr2 = _mid(a0, a1, d0, d1, hr1, r(g1), r(b1), wl2, wr2, r(bl2))
    agg2 = spmm(hlp2, src2d, dst2d)
    a0, a1 = _split(agg2)
    hlp3, hr3 = _mid(a0, a1, d0, d1, hr2, r(g2), r(b2), wl3, wr3, r(bl3))
    agg3 = spmm(hlp3, src2d, dst2d)
    a0, a1 = _split(agg3)
    node_emb, graph_emb = _final(a0, a1, d0, d1, hr3, r(g3), r(b3), batch3d)
    return (graph_emb, node_emb)


# speedup vs baseline: 10.8423x; 1.1535x over previous
import functools

import jax
import jax.numpy as jnp
from jax import lax
from jax.experimental import pallas as pl
from jax.experimental.pallas import tpu as pltpu
from jax.experimental.pallas import tpu_sc as plsc

N = 10000
E = 320000
D = 128
H = 64
G = 8
EPS = 1e-5

NC = 2
NS = 16
NW = NC * NS
CH = 64
NCHUNK = 5120
EPAD = NCHUNK * CH
CPW = NCHUNK // NW
HALF = CPW // 4
NB = 4
NP = 10240
RPS = NP // NS
W = 128


@functools.lru_cache(maxsize=None)
def _make_spmm():
    mesh = plsc.VectorSubcoreMesh(core_axis_name="c", subcore_axis_name="s",
                                  num_cores=NC, num_subcores=NS)
    out_type = jax.ShapeDtypeStruct((2 * NP, W), jnp.float32)
    scratch = [
        pltpu.VMEM((HALF, CH), jnp.int32),
        pltpu.VMEM((HALF, CH), jnp.int32),
    ]
    scratch += [pltpu.VMEM((CH, W), jnp.float32) for _ in range(NB)]
    scratch += [pltpu.SemaphoreType.DMA for _ in range(2 * NB)]
    scratch += [pltpu.VMEM_SHARED((NP, W), jnp.float32)]

    def body(hlp, src_i, dst_i, agg_o, src_v, dst_v, *rest):
        rows = rest[:NB]
        sems = rest[NB:NB + 2 * NB]
        agg_sh = rest[NB + 2 * NB]
        sg = sems[:NB]
        ss = sems[NB:]

        cid = lax.axis_index("c")
        sid = lax.axis_index("s")
        w = cid * NS + sid

        @pl.loop(0, CH)
        def _(r):
            for c in range(W // 16):
                rows[0][r, pl.ds(c * 16, 16)] = jnp.zeros((16,), jnp.float32)

        @pl.loop(0, RPS // CH)
        def _(i):
            pltpu.sync_copy(rows[0], agg_sh.at[pl.ds(sid * RPS + i * CH, CH)])

        plsc.subcore_barrier()

        for phase in range(CPW // HALF):
            base_row = w * CPW + phase * HALF
            pltpu.sync_copy(src_i.at[pl.ds(base_row, HALF)], src_v)
            pltpu.sync_copy(dst_i.at[pl.ds(base_row, HALF)], dst_v)

            for b in range(NB):
                pltpu.async_copy(hlp.at[src_v.at[b]], rows[b], sg[b])

            @pl.loop(0, HALF, step=NB)
            def _(base):
                for b in range(NB):
                    k = base + b
                    pltpu.make_async_copy(hlp.at[src_v.at[0]], rows[b],
                                          sg[b]).wait()
                    pltpu.async_copy(rows[b], agg_sh.at[dst_v.at[k]], ss[b],
                                     add=True)
                for b in range(NB):
                    nk = base + NB + b
                    pltpu.make_async_copy(rows[b], agg_sh.at[dst_v.at[0]],
                                          ss[b]).wait()

                    @pl.when(nk < HALF)
                    def _():
                        pltpu.async_copy(hlp.at[src_v.at[nk]], rows[b], sg[b])

        plsc.subcore_barrier()

        pltpu.sync_copy(agg_sh.at[pl.ds(sid * RPS, RPS)],
                        agg_o.at[pl.ds(cid * NP + sid * RPS, RPS)])

    return pl.kernel(body, out_type=out_type, mesh=mesh,
                     scratch_types=scratch)


BN = 2000
NBLK = N // BN


def _dot_t(a, w):
    return lax.dot_general(a, w, (((1,), (1,)), ((), ())),
                           preferred_element_type=jnp.float32)


def _pad_cols(hl, deg_lane):
    if deg_lane:
        extra = (lax.broadcasted_iota(jnp.int32, (BN, W - H), 1) == 0
                 ).astype(jnp.float32)
    else:
        extra = jnp.zeros((BN, W - H), jnp.float32)
    return jnp.concatenate([hl, extra], axis=1)


def _proj0_body(x_ref, wl_ref, wr_ref, bl_ref, hlp_ref, hr_ref):
    xb = x_ref[...]
    hlp_ref[...] = _pad_cols(_dot_t(xb, wl_ref[...]), True)
    hr_ref[...] = _dot_t(xb, wr_ref[...]) + bl_ref[...]


def _norm_block(a0, a1, d0, d1, hr, g, b):
    deg = jnp.maximum(d0[...][:, 0:1] + d1[...][:, 0:1], 1.0)
    t = (a0[...][:, :H] + a1[...][:, :H]) / deg + hr[...]
    m = jnp.mean(t, axis=-1, keepdims=True)
    v = jnp.mean((t - m) ** 2, axis=-1, keepdims=True)
    h = (t - m) * lax.rsqrt(v + EPS) * g[...] + b[...]
    return jnp.maximum(h, 0.0)


def _mid_body(a0, a1, d0, d1, hr, g, b, wl, wr, bln, hlp_o, hr_o):
    h = _norm_block(a0, a1, d0, d1, hr, g, b)
    hlp_o[...] = _pad_cols(_dot_t(h, wl[...]), False)
    hr_o[...] = _dot_t(h, wr[...]) + bln[...]


def _final_body(a0, a1, d0, d1, hr, g, b, batch_ref, node_o, graph_o,
                sums_acc, cnt_acc):
    i = pl.program_id(0)
    h = _norm_block(a0, a1, d0, d1, hr, g, b)
    node_o[...] = h
    bi = batch_ref[0]
    oh = (lax.broadcasted_iota(jnp.int32, (G, BN), 0) == bi
          ).astype(jnp.float32)
    s = lax.dot_general(oh, h, (((1,), (0,)), ((), ())),
                        preferred_element_type=jnp.float32)
    c = jnp.sum(oh, axis=1, keepdims=True)

    @pl.when(i == 0)
    def _():
        sums_acc[...] = jnp.zeros_like(sums_acc)
        cnt_acc[...] = jnp.zeros_like(cnt_acc)

    sums_acc[...] += s
    cnt_acc[...] += jnp.broadcast_to(c, (G, H))

    @pl.when(i == NBLK - 1)
    def _():
        graph_o[...] = sums_acc[...] / jnp.maximum(cnt_acc[...], 1.0)


def _row_spec(shape):
    return pl.BlockSpec((BN,) + shape[1:],
                        lambda i: (i,) + (0,) * (len(shape) - 1))


def _full_spec(shape):
    return pl.BlockSpec(shape, lambda i: (0,) * len(shape))


def _proj0(x, wl, wr, bl):
    return pl.pallas_call(
        _proj0_body,
        grid=(NBLK,),
        in_specs=[_row_spec((N, D)), _full_spec((H, D)), _full_spec((H, D)),
                  _full_spec((1, H))],
        out_specs=[_row_spec((N, W)), _row_spec((N, H))],
        out_shape=[jax.ShapeDtypeStruct((N, W), jnp.float32),
                   jax.ShapeDtypeStruct((N, H), jnp.float32)],
    )(x, wl, wr, bl)


def _mid(a0, a1, d0, d1, hr, g, b, wl, wr, bln):
    return pl.pallas_call(
        _mid_body,
        grid=(NBLK,),
        in_specs=[_row_spec((N, W))] * 2 + [_row_spec((N, 16))] * 2 + [
            _row_spec((N, H)), _full_spec((1, H)), _full_spec((1, H)),
            _full_spec((H, H)), _full_spec((H, H)), _full_spec((1, H))],
        out_specs=[_row_spec((N, W)), _row_spec((N, H))],
        out_shape=[jax.ShapeDtypeStruct((N, W), jnp.float32),
                   jax.ShapeDtypeStruct((N, H), jnp.float32)],
    )(a0, a1, d0, d1, hr, g, b, wl, wr, bln)


def _final(a0, a1, d0, d1, hr, g, b, batch3d):
    return pl.pallas_call(
        _final_body,
        grid=(NBLK,),
        in_specs=[_row_spec((N, W))] * 2 + [_row_spec((N, 16))] * 2 + [
            _row_spec((N, H)), _full_spec((1, H)), _full_spec((1, H)),
            pl.BlockSpec((1, 1, BN), lambda i: (i, 0, 0))],
        out_specs=[_row_spec((N, H)), _full_spec((G, H))],
        out_shape=[jax.ShapeDtypeStruct((N, H), jnp.float32),
                   jax.ShapeDtypeStruct((G, H), jnp.float32)],
        scratch_shapes=[pltpu.VMEM((G, H), jnp.float32),
                        pltpu.VMEM((G, H), jnp.float32)],
    )(a0, a1, d0, d1, hr, g, b, batch3d)


def _split(parts):
    return parts[:N], parts[NP:NP + N]


def kernel(x, edge_index, batch, params):
    pad = EPAD - E
    lanes = jnp.arange(pad, dtype=jnp.int32) % CH
    src2d = jnp.concatenate(
        [edge_index[0], lanes]).reshape(NCHUNK, CH)
    dst2d = jnp.concatenate(
        [edge_index[1], N + lanes]).reshape(NCHUNK, CH)
    batch3d = batch.reshape(NBLK, 1, BN)
    (wl1, bl1, wr1, g1, b1) = params[0]
    (wl2, bl2, wr2, g2, b2) = params[1]
    (wl3, bl3, wr3, g3, b3) = params[2]
    r = lambda v: v.reshape(1, H)
    spmm = _make_spmm()

    hlp1, hr1 = _proj0(x, wl1, wr1, r(bl1))
    agg1 = spmm(hlp1, src2d, dst2d)
    a0, a1 = _split(agg1)
    d0 = agg1[:N, H:H + 16]
    d1 = agg1[NP:NP + N, H:H + 16]
    hlp2, hr2 = _mid(a0, a1, d0, d1, hr1, r(g1), r(b1), wl2, wr2, r(bl2))
    agg2 = spmm(hlp2, src2d, dst2d)
    a0, a1 = _split(agg2)
    hlp3, hr3 = _mid(a0, a1, d0, d1, hr2, r(g2), r(b2), wl3, wr3, r(bl3))
    agg3 = spmm(hlp3, src2d, dst2d)
    a0, a1 = _split(agg3)
    node_emb, graph_emb = _final(a0, a1, d0, d1, hr3, r(g3), r(b3), batch3d)
    return (graph_emb, node_emb)


# speedup vs baseline: 11.7256x; 1.0815x over previous
import functools

import jax
import jax.numpy as jnp
from jax import lax
from jax.experimental import pallas as pl
from jax.experimental.pallas import tpu as pltpu
from jax.experimental.pallas import tpu_sc as plsc

N = 10000
E = 320000
D = 128
H = 64
G = 8
EPS = 1e-5

NC = 2
NS = 16
NW = NC * NS
CH = 64
NCHUNK = 5120
EPAD = NCHUNK * CH
CPW = NCHUNK // NW
HALF = CPW // 4
NB = 4
NP = 10240
RPS = NP // NS
W = 128


@functools.lru_cache(maxsize=None)
def _make_spmm():
    mesh = plsc.VectorSubcoreMesh(core_axis_name="c", subcore_axis_name="s",
                                  num_cores=NC, num_subcores=NS)
    out_type = jax.ShapeDtypeStruct((2, NP, W), jnp.float32)
    scratch = [
        pltpu.VMEM((HALF, CH), jnp.int32),
        pltpu.VMEM((HALF, CH), jnp.int32),
    ]
    scratch += [pltpu.VMEM((CH, W), jnp.float32) for _ in range(NB)]
    scratch += [pltpu.SemaphoreType.DMA for _ in range(2 * NB)]
    scratch += [pltpu.VMEM_SHARED((NP, W), jnp.float32)]

    def body(hlp, src_i, dst_i, agg_o, src_v, dst_v, *rest):
        rows = rest[:NB]
        sems = rest[NB:NB + 2 * NB]
        agg_sh = rest[NB + 2 * NB]
        sg = sems[:NB]
        ss = sems[NB:]

        cid = lax.axis_index("c")
        sid = lax.axis_index("s")
        w = cid * NS + sid

        @pl.loop(0, CH)
        def _(r):
            for c in range(W // 16):
                rows[0][r, pl.ds(c * 16, 16)] = jnp.zeros((16,), jnp.float32)

        @pl.loop(0, RPS // CH)
        def _(i):
            pltpu.sync_copy(rows[0], agg_sh.at[pl.ds(sid * RPS + i * CH, CH)])

        plsc.subcore_barrier()

        for phase in range(CPW // HALF):
            base_row = w * CPW + phase * HALF
            pltpu.sync_copy(src_i.at[pl.ds(base_row, HALF)], src_v)
            pltpu.sync_copy(dst_i.at[pl.ds(base_row, HALF)], dst_v)

            for b in range(NB):
                pltpu.async_copy(hlp.at[src_v.at[b]], rows[b], sg[b])

            @pl.loop(0, HALF, step=NB)
            def _(base):
                for b in range(NB):
                    k = base + b
                    pltpu.make_async_copy(hlp.at[src_v.at[0]], rows[b],
                                          sg[b]).wait()
                    pltpu.async_copy(rows[b], agg_sh.at[dst_v.at[k]], ss[b],
                                     add=True)
                for b in range(NB):
                    nk = base + NB + b
                    pltpu.make_async_copy(rows[b], agg_sh.at[dst_v.at[0]],
                                          ss[b]).wait()

                    @pl.when(nk < HALF)
                    def _():
                        pltpu.async_copy(hlp.at[src_v.at[nk]], rows[b], sg[b])

        plsc.subcore_barrier()

        pltpu.sync_copy(agg_sh.at[pl.ds(sid * RPS, RPS)],
                        agg_o.at[cid, pl.ds(sid * RPS, RPS)])

    return pl.kernel(body, out_type=out_type, mesh=mesh,
                     scratch_types=scratch)


BN = 2000
NBLK = N // BN


def _dot_t(a, w):
    return lax.dot_general(a, w, (((1,), (1,)), ((), ())),
                           preferred_element_type=jnp.float32)


def _pad_cols(hl):
    extra = (lax.broadcasted_iota(jnp.int32, (BN, W - H), 1) == 0
             ).astype(jnp.float32)
    return jnp.concatenate([hl, extra], axis=1)


def _proj0_body(x_ref, wl_ref, wr_ref, bl_ref, hlp_ref, hr_ref):
    xb = x_ref[...]
    hlp_ref[...] = _pad_cols(_dot_t(xb, wl_ref[...]))
    hr_ref[...] = _dot_t(xb, wr_ref[...]) + bl_ref[...]


def _norm_block(a0, a1, hr, g, b):
    p0 = a0[0]
    p1 = a1[0]
    deg = jnp.maximum(p0[:, H:H + 1] + p1[:, H:H + 1], 1.0)
    t = (p0[:, :H] + p1[:, :H]) / deg + hr[...]
    m = jnp.mean(t, axis=-1, keepdims=True)
    v = jnp.mean((t - m) ** 2, axis=-1, keepdims=True)
    h = (t - m) * lax.rsqrt(v + EPS) * g[...] + b[...]
    return jnp.maximum(h, 0.0)


def _mid_body(a0, a1, hr, g, b, wl, wr, bln, hlp_o, hr_o):
    h = _norm_block(a0, a1, hr, g, b)
    hlp_o[...] = _pad_cols(_dot_t(h, wl[...]))
    hr_o[...] = _dot_t(h, wr[...]) + bln[...]


def _final_body(a0, a1, hr, g, b, batch_ref, node_o, graph_o,
                sums_acc, cnt_acc):
    i = pl.program_id(0)
    h = _norm_block(a0, a1, hr, g, b)
    node_o[...] = h
    bi = batch_ref[0]
    oh = (lax.broadcasted_iota(jnp.int32, (G, BN), 0) == bi
          ).astype(jnp.float32)
    s = lax.dot_general(oh, h, (((1,), (0,)), ((), ())),
                        preferred_element_type=jnp.float32)
    c = jnp.sum(oh, axis=1, keepdims=True)

    @pl.when(i == 0)
    def _():
        sums_acc[...] = jnp.zeros_like(sums_acc)
        cnt_acc[...] = jnp.zeros_like(cnt_acc)

    sums_acc[...] += s
    cnt_acc[...] += jnp.broadcast_to(c, (G, H))

    @pl.when(i == NBLK - 1)
    def _():
        graph_o[...] = sums_acc[...] / jnp.maximum(cnt_acc[...], 1.0)


def _row_spec(shape):
    return pl.BlockSpec((BN,) + shape[1:],
                        lambda i: (i,) + (0,) * (len(shape) - 1))


def _full_spec(shape):
    return pl.BlockSpec(shape, lambda i: (0,) * len(shape))


def _proj0(x, wl, wr, bl):
    return pl.pallas_call(
        _proj0_body,
        grid=(NBLK,),
        in_specs=[_row_spec((N, D)), _full_spec((H, D)), _full_spec((H, D)),
                  _full_spec((1, H))],
        out_specs=[_row_spec((N, W)), _row_spec((N, H))],
        out_shape=[jax.ShapeDtypeStruct((N, W), jnp.float32),
                   jax.ShapeDtypeStruct((N, H), jnp.float32)],
    )(x, wl, wr, bl)


_PART0 = pl.BlockSpec((1, BN, W), lambda i: (0, i, 0))
_PART1 = pl.BlockSpec((1, BN, W), lambda i: (1, i, 0))


def _mid(agg, hr, g, b, wl, wr, bln):
    return pl.pallas_call(
        _mid_body,
        grid=(NBLK,),
        in_specs=[_PART0, _PART1,
                  _row_spec((N, H)), _full_spec((1, H)), _full_spec((1, H)),
                  _full_spec((H, H)), _full_spec((H, H)), _full_spec((1, H))],
        out_specs=[_row_spec((N, W)), _row_spec((N, H))],
        out_shape=[jax.ShapeDtypeStruct((N, W), jnp.float32),
                   jax.ShapeDtypeStruct((N, H), jnp.float32)],
    )(agg, agg, hr, g, b, wl, wr, bln)


def _final(agg, hr, g, b, batch3d):
    return pl.pallas_call(
        _final_body,
        grid=(NBLK,),
        in_specs=[_PART0, _PART1,
                  _row_spec((N, H)), _full_spec((1, H)), _full_spec((1, H)),
                  pl.BlockSpec((1, 1, BN), lambda i: (i, 0, 0))],
        out_specs=[_row_spec((N, H)), _full_spec((G, H))],
        out_shape=[jax.ShapeDtypeStruct((N, H), jnp.float32),
                   jax.ShapeDtypeStruct((G, H), jnp.float32)],
        scratch_shapes=[pltpu.VMEM((G, H), jnp.float32),
                        pltpu.VMEM((G, H), jnp.float32)],
    )(agg, agg, hr, g, b, batch3d)


def kernel(x, edge_index, batch, params):
    pad = EPAD - E
    lanes = jnp.arange(pad, dtype=jnp.int32) % CH
    src2d = jnp.concatenate(
        [edge_index[0], lanes]).reshape(NCHUNK, CH)
    dst2d = jnp.concatenate(
        [edge_index[1], N + lanes]).reshape(NCHUNK, CH)
    batch3d = batch.reshape(NBLK, 1, BN)
    (wl1, bl1, wr1, g1, b1) = params[0]
    (wl2, bl2, wr2, g2, b2) = params[1]
    (wl3, bl3, wr3, g3, b3) = params[2]
    r = lambda v: v.reshape(1, H)
    spmm = _make_spmm()

    hlp1, hr1 = _proj0(x, wl1, wr1, r(bl1))
    agg1 = spmm(hlp1, src2d, dst2d)
    hlp2, hr2 = _mid(agg1, hr1, r(g1), r(b1), wl2, wr2, r(bl2))
    agg2 = spmm(hlp2, src2d, dst2d)
    hlp3, hr3 = _mid(agg2, hr2, r(g2), r(b2), wl3, wr3, r(bl3))
    agg3 = spmm(hlp3, src2d, dst2d)
    node_emb, graph_emb = _final(agg3, hr3, r(g3), r(b3), batch3d)
    return (graph_emb, node_emb)


# speedup vs baseline: 11.9193x; 1.0165x over previous
import functools

import jax
import jax.numpy as jnp
from jax import lax
from jax.experimental import pallas as pl
from jax.experimental.pallas import tpu as pltpu
from jax.experimental.pallas import tpu_sc as plsc

N = 10000
E = 320000
D = 128
H = 64
G = 8
EPS = 1e-5

NC = 2
NS = 16
NW = NC * NS
CH = 64
NCHUNK = 5120
EPAD = NCHUNK * CH
CPW = NCHUNK // NW
HALF = CPW // 4
NB = 4
NP = 10240
RPS = NP // NS
W = 128


@functools.lru_cache(maxsize=None)
def _make_spmm():
    mesh = plsc.VectorSubcoreMesh(core_axis_name="c", subcore_axis_name="s",
                                  num_cores=NC, num_subcores=NS)
    out_type = jax.ShapeDtypeStruct((2, NP, W), jnp.float32)
    scratch = [
        pltpu.VMEM((HALF, CH), jnp.int32),
        pltpu.VMEM((HALF, CH), jnp.int32),
    ]
    scratch += [pltpu.VMEM((CH, W), jnp.float32) for _ in range(NB)]
    scratch += [pltpu.SemaphoreType.DMA for _ in range(2 * NB)]
    scratch += [pltpu.VMEM_SHARED((NP, W), jnp.float32)]

    def body(hlp, src_i, dst_i, agg_o, src_v, dst_v, *rest):
        rows = rest[:NB]
        sems = rest[NB:NB + 2 * NB]
        agg_sh = rest[NB + 2 * NB]
        sg = sems[:NB]
        ss = sems[NB:]

        cid = lax.axis_index("c")
        sid = lax.axis_index("s")
        w = cid * NS + sid

        @pl.loop(0, CH)
        def _(r):
            for c in range(W // 16):
                rows[0][r, pl.ds(c * 16, 16)] = jnp.zeros((16,), jnp.float32)

        for i in range(RPS // CH):
            pltpu.async_copy(rows[0], agg_sh.at[pl.ds(sid * RPS + i * CH, CH)],
                             sg[0])
        for i in range(RPS // CH):
            pltpu.make_async_copy(rows[0], agg_sh.at[pl.ds(sid * RPS, CH)],
                                  sg[0]).wait()

        plsc.subcore_barrier()

        for phase in range(CPW // HALF):
            base_row = w * CPW + phase * HALF
            pltpu.async_copy(src_i.at[pl.ds(base_row, HALF)], src_v, sg[0])
            pltpu.async_copy(dst_i.at[pl.ds(base_row, HALF)], dst_v, sg[0])
            pltpu.make_async_copy(src_i.at[pl.ds(base_row, HALF)], src_v,
                                  sg[0]).wait()
            pltpu.make_async_copy(dst_i.at[pl.ds(base_row, HALF)], dst_v,
                                  sg[0]).wait()

            for b in range(NB):
                pltpu.async_copy(hlp.at[src_v.at[b]], rows[b], sg[b])

            @pl.loop(0, HALF, step=NB)
            def _(base):
                for b in range(NB):
                    k = base + b
                    pltpu.make_async_copy(hlp.at[src_v.at[0]], rows[b],
                                          sg[b]).wait()
                    pltpu.async_copy(rows[b], agg_sh.at[dst_v.at[k]], ss[b],
                                     add=True)
                for b in range(NB):
                    nk = base + NB + b
                    pltpu.make_async_copy(rows[b], agg_sh.at[dst_v.at[0]],
                                          ss[b]).wait()

                    @pl.when(nk < HALF)
                    def _():
                        pltpu.async_copy(hlp.at[src_v.at[nk]], rows[b], sg[b])

        plsc.subcore_barrier()

        pltpu.sync_copy(agg_sh.at[pl.ds(sid * RPS, RPS)],
                        agg_o.at[cid, pl.ds(sid * RPS, RPS)])

    return pl.kernel(body, out_type=out_type, mesh=mesh,
                     scratch_types=scratch)


BN = 2000
NBLK = N // BN


def _dot_t(a, w):
    return lax.dot_general(a, w, (((1,), (1,)), ((), ())),
                           preferred_element_type=jnp.float32)


def _pad_cols(hl):
    extra = (lax.broadcasted_iota(jnp.int32, (BN, W - H), 1) == 0
             ).astype(jnp.float32)
    return jnp.concatenate([hl, extra], axis=1)


def _proj0_body(x_ref, wl_ref, wr_ref, bl_ref, hlp_ref, hr_ref):
    xb = x_ref[...]
    hlp_ref[...] = _pad_cols(_dot_t(xb, wl_ref[...]))
    hr_ref[...] = _dot_t(xb, wr_ref[...]) + bl_ref[...]


def _norm_block(a0, a1, hr, g, b):
    p0 = a0[0]
    p1 = a1[0]
    deg = jnp.maximum(p0[:, H:H + 1] + p1[:, H:H + 1], 1.0)
    t = (p0[:, :H] + p1[:, :H]) / deg + hr[...]
    m = jnp.mean(t, axis=-1, keepdims=True)
    v = jnp.mean((t - m) ** 2, axis=-1, keepdims=True)
    h = (t - m) * lax.rsqrt(v + EPS) * g[...] + b[...]
    return jnp.maximum(h, 0.0)


def _mid_body(a0, a1, hr, g, b, wl, wr, bln, hlp_o, hr_o):
    h = _norm_block(a0, a1, hr, g, b)
    hlp_o[...] = _pad_cols(_dot_t(h, wl[...]))
    hr_o[...] = _dot_t(h, wr[...]) + bln[...]


def _final_body(a0, a1, hr, g, b, batch_ref, node_o, graph_o,
                sums_acc, cnt_acc):
    i = pl.program_id(0)
    h = _norm_block(a0, a1, hr, g, b)
    node_o[...] = h
    bi = batch_ref[0]
    oh = (lax.broadcasted_iota(jnp.int32, (G, BN), 0) == bi
          ).astype(jnp.float32)
    s = lax.dot_general(oh, h, (((1,), (0,)), ((), ())),
                        preferred_element_type=jnp.float32)
    c = jnp.sum(oh, axis=1, keepdims=True)

    @pl.when(i == 0)
    def _():
        sums_acc[...] = jnp.zeros_like(sums_acc)
        cnt_acc[...] = jnp.zeros_like(cnt_acc)

    sums_acc[...] += s
    cnt_acc[...] += jnp.broadcast_to(c, (G, H))

    @pl.when(i == NBLK - 1)
    def _():
        graph_o[...] = sums_acc[...] / jnp.maximum(cnt_acc[...], 1.0)


def _row_spec(shape):
    return pl.BlockSpec((BN,) + shape[1:],
                        lambda i: (i,) + (0,) * (len(shape) - 1))


def _full_spec(shape):
    return pl.BlockSpec(shape, lambda i: (0,) * len(shape))


def _proj0(x, wl, wr, bl):
    return pl.pallas_call(
        _proj0_body,
        grid=(NBLK,),
        in_specs=[_row_spec((N, D)), _full_spec((H, D)), _full_spec((H, D)),
                  _full_spec((1, H))],
        out_specs=[_row_spec((N, W)), _row_spec((N, H))],
        out_shape=[jax.ShapeDtypeStruct((N, W), jnp.float32),
                   jax.ShapeDtypeStruct((N, H), jnp.float32)],
    )(x, wl, wr, bl)


_PART0 = pl.BlockSpec((1, BN, W), lambda i: (0, i, 0))
_PART1 = pl.BlockSpec((1, BN, W), lambda i: (1, i, 0))


def _mid(agg, hr, g, b, wl, wr, bln):
    return pl.pallas_call(
        _mid_body,
        grid=(NBLK,),
        in_specs=[_PART0, _PART1,
                  _row_spec((N, H)), _full_spec((1, H)), _full_spec((1, H)),
                  _full_spec((H, H)), _full_spec((H, H)), _full_spec((1, H))],
        out_specs=[_row_spec((N, W)), _row_spec((N, H))],
        out_shape=[jax.ShapeDtypeStruct((N, W), jnp.float32),
                   jax.ShapeDtypeStruct((N, H), jnp.float32)],
    )(agg, agg, hr, g, b, wl, wr, bln)


def _final(agg, hr, g, b, batch3d):
    return pl.pallas_call(
        _final_body,
        grid=(NBLK,),
        in_specs=[_PART0, _PART1,
                  _row_spec((N, H)), _full_spec((1, H)), _full_spec((1, H)),
                  pl.BlockSpec((1, 1, BN), lambda i: (i, 0, 0))],
        out_specs=[_row_spec((N, H)), _full_spec((G, H))],
        out_shape=[jax.ShapeDtypeStruct((N, H), jnp.float32),
                   jax.ShapeDtypeStruct((G, H), jnp.float32)],
        scratch_shapes=[pltpu.VMEM((G, H), jnp.float32),
                        pltpu.VMEM((G, H), jnp.float32)],
    )(agg, agg, hr, g, b, batch3d)


def kernel(x, edge_index, batch, params):
    pad = EPAD - E
    lanes = jnp.arange(pad, dtype=jnp.int32) % CH
    src2d = jnp.concatenate(
        [edge_index[0], lanes]).reshape(NCHUNK, CH)
    dst2d = jnp.concatenate(
        [edge_index[1], N + lanes]).reshape(NCHUNK, CH)
    batch3d = batch.reshape(NBLK, 1, BN)
    (wl1, bl1, wr1, g1, b1) = params[0]
    (wl2, bl2, wr2, g2, b2) = params[1]
    (wl3, bl3, wr3, g3, b3) = params[2]
    r = lambda v: v.reshape(1, H)
    spmm = _make_spmm()

    hlp1, hr1 = _proj0(x, wl1, wr1, r(bl1))
    agg1 = spmm(hlp1, src2d, dst2d)
    hlp2, hr2 = _mid(agg1, hr1, r(g1), r(b1), wl2, wr2, r(bl2))
    agg2 = spmm(hlp2, src2d, dst2d)
    hlp3, hr3 = _mid(agg2, hr2, r(g2), r(b2), wl3, wr3, r(bl3))
    agg3 = spmm(hlp3, src2d, dst2d)
    node_emb, graph_emb = _final(agg3, hr3, r(g3), r(b3), batch3d)
    return (graph_emb, node_emb)


# speedup vs baseline: 12.3084x; 1.0326x over previous
import functools

import jax
import jax.numpy as jnp
from jax import lax
from jax.experimental import pallas as pl
from jax.experimental.pallas import tpu as pltpu
from jax.experimental.pallas import tpu_sc as plsc

N = 10000
E = 320000
D = 128
H = 64
G = 8
EPS = 1e-5

NC = 2
NS = 16
NW = NC * NS
CH = 64
NCHUNK = 5120
EPAD = NCHUNK * CH
CPW = NCHUNK // NW
HALF = CPW // 4
NB = 4
NP = 10240
RPS = NP // NS
W = 128
EPW = EPAD // NW
EPP = HALF * CH


@functools.lru_cache(maxsize=None)
def _make_spmm():
    mesh = plsc.VectorSubcoreMesh(core_axis_name="c", subcore_axis_name="s",
                                  num_cores=NC, num_subcores=NS)
    out_type = jax.ShapeDtypeStruct((2, NP, W), jnp.float32)
    scratch = [
        pltpu.VMEM((EPP,), jnp.int32),
        pltpu.VMEM((EPP,), jnp.int32),
    ]
    scratch += [pltpu.VMEM((CH, W), jnp.float32) for _ in range(NB)]
    scratch += [pltpu.SemaphoreType.DMA for _ in range(2 * NB)]
    scratch += [pltpu.VMEM_SHARED((NP, W), jnp.float32)]

    def body(hlp, eflat, psrc, pdst, agg_o, src_v, dst_v, *rest):
        rows = rest[:NB]
        sems = rest[NB:NB + 2 * NB]
        agg_sh = rest[NB + 2 * NB]
        sg = sems[:NB]
        ss = sems[NB:]

        cid = lax.axis_index("c")
        sid = lax.axis_index("s")
        w = cid * NS + sid

        @pl.loop(0, CH)
        def _(r):
            for c in range(W // 16):
                rows[0][r, pl.ds(c * 16, 16)] = jnp.zeros((16,), jnp.float32)

        for i in range(RPS // CH):
            pltpu.async_copy(rows[0], agg_sh.at[pl.ds(sid * RPS + i * CH, CH)],
                             sg[0])
        for i in range(RPS // CH):
            pltpu.make_async_copy(rows[0], agg_sh.at[pl.ds(sid * RPS, CH)],
                                  sg[0]).wait()

        plsc.subcore_barrier()

        def chunk_idx(idx_v, k):
            return idx_v.at[pl.ds(pl.multiple_of(k * CH, CH), CH)]

        for phase in range(CPW // HALF):
            off = pl.multiple_of(w * EPW + phase * EPP, EPP)
            if phase == 0:
                pltpu.async_copy(eflat.at[pl.ds(off, EPP)], src_v, sg[0])
                pltpu.async_copy(eflat.at[pl.ds(E + off, EPP)], dst_v, sg[1])
            else:
                @pl.when(w < NW - 1)
                def _():
                    pltpu.async_copy(eflat.at[pl.ds(off, EPP)], src_v, sg[0])
                    pltpu.async_copy(eflat.at[pl.ds(E + off, EPP)], dst_v,
                                     sg[1])

                @pl.when(w == NW - 1)
                def _():
                    pltpu.async_copy(psrc.at[pl.ds((phase - 1) * EPP, EPP)],
                                     src_v, sg[0])
                    pltpu.async_copy(pdst.at[pl.ds((phase - 1) * EPP, EPP)],
                                     dst_v, sg[1])
            pltpu.make_async_copy(eflat.at[pl.ds(off, EPP)], src_v,
                                  sg[0]).wait()
            pltpu.make_async_copy(eflat.at[pl.ds(off, EPP)], dst_v,
                                  sg[1]).wait()

            for b in range(NB):
                pltpu.async_copy(hlp.at[chunk_idx(src_v, b)], rows[b], sg[b])

            @pl.loop(0, HALF, step=NB)
            def _(base):
                for b in range(NB):
                    k = base + b
                    pltpu.make_async_copy(hlp.at[chunk_idx(src_v, 0)], rows[b],
                                          sg[b]).wait()
                    pltpu.async_copy(rows[b], agg_sh.at[chunk_idx(dst_v, k)],
                                     ss[b], add=True)
                for b in range(NB):
                    nk = base + NB + b
                    pltpu.make_async_copy(rows[b], agg_sh.at[chunk_idx(dst_v, 0)],
                                          ss[b]).wait()

                    @pl.when(nk < HALF)
                    def _():
                        pltpu.async_copy(hlp.at[chunk_idx(src_v, nk)], rows[b],
                                         sg[b])

        plsc.subcore_barrier()

        pltpu.sync_copy(agg_sh.at[pl.ds(sid * RPS, RPS)],
                        agg_o.at[cid, pl.ds(sid * RPS, RPS)])

    return pl.kernel(body, out_type=out_type, mesh=mesh,
                     scratch_types=scratch)


BN = 2000
NBLK = N // BN


def _dot_t(a, w):
    return lax.dot_general(a, w, (((1,), (1,)), ((), ())),
                           preferred_element_type=jnp.float32)


def _pad_cols(hl):
    extra = (lax.broadcasted_iota(jnp.int32, (BN, W - H), 1) == 0
             ).astype(jnp.float32)
    return jnp.concatenate([hl, extra], axis=1)


def _proj0_body(x_ref, wl_ref, wr_ref, bl_ref, hlp_ref, hr_ref):
    xb = x_ref[...]
    hlp_ref[...] = _pad_cols(_dot_t(xb, wl_ref[...]))
    hr_ref[...] = _dot_t(xb, wr_ref[...]) + bl_ref[...]


def _norm_block(a0, a1, hr, g, b):
    p0 = a0[0]
    p1 = a1[0]
    deg = jnp.maximum(p0[:, H:H + 1] + p1[:, H:H + 1], 1.0)
    t = (p0[:, :H] + p1[:, :H]) / deg + hr[...]
    m = jnp.mean(t, axis=-1, keepdims=True)
    v = jnp.mean((t - m) ** 2, axis=-1, keepdims=True)
    h = (t - m) * lax.rsqrt(v + EPS) * g[...] + b[...]
    return jnp.maximum(h, 0.0)


def _mid_body(a0, a1, hr, g, b, wl, wr, bln, hlp_o, hr_o):
    h = _norm_block(a0, a1, hr, g, b)
    hlp_o[...] = _pad_cols(_dot_t(h, wl[...]))
    hr_o[...] = _dot_t(h, wr[...]) + bln[...]


def _final_body(a0, a1, hr, g, b, batch_ref, node_o, graph_o,
                sums_acc, cnt_acc):
    i = pl.program_id(0)
    h = _norm_block(a0, a1, hr, g, b)
    node_o[...] = h
    bi = batch_ref[0]
    oh = (lax.broadcasted_iota(jnp.int32, (G, BN), 0) == bi
          ).astype(jnp.float32)
    s = lax.dot_general(oh, h, (((1,), (0,)), ((), ())),
                        preferred_element_type=jnp.float32)
    c = jnp.sum(oh, axis=1, keepdims=True)

    @pl.when(i == 0)
    def _():
        sums_acc[...] = jnp.zeros_like(sums_acc)
        cnt_acc[...] = jnp.zeros_like(cnt_acc)

    sums_acc[...] += s
    cnt_acc[...] += jnp.broadcast_to(c, (G, H))

    @pl.when(i == NBLK - 1)
    def _():
        graph_o[...] = sums_acc[...] / jnp.maximum(cnt_acc[...], 1.0)


def _row_spec(shape):
    return pl.BlockSpec((BN,) + shape[1:],
                        lambda i: (i,) + (0,) * (len(shape) - 1))


def _full_spec(shape):
    return pl.BlockSpec(shape, lambda i: (0,) * len(shape))


def _proj0(x, wl, wr, bl):
    return pl.pallas_call(
        _proj0_body,
        grid=(NBLK,),
        in_specs=[_row_spec((N, D)), _full_spec((H, D)), _full_spec((H, D)),
                  _full_spec((1, H))],
        out_specs=[_row_spec((N, W)), _row_spec((N, H))],
        out_shape=[jax.ShapeDtypeStruct((N, W), jnp.float32),
                   jax.ShapeDtypeStruct((N, H), jnp.float32)],
    )(x, wl, wr, bl)


_PART0 = pl.BlockSpec((1, BN, W), lambda i: (0, i, 0))
_PART1 = pl.BlockSpec((1, BN, W), lambda i: (1, i, 0))


def _mid(agg, hr, g, b, wl, wr, bln):
    return pl.pallas_call(
        _mid_body,
        grid=(NBLK,),
        in_specs=[_PART0, _PART1,
                  _row_spec((N, H)), _full_spec((1, H)), _full_spec((1, H)),
                  _full_spec((H, H)), _full_spec((H, H)), _full_spec((1, H))],
        out_specs=[_row_spec((N, W)), _row_spec((N, H))],
        out_shape=[jax.ShapeDtypeStruct((N, W), jnp.float32),
                   jax.ShapeDtypeStruct((N, H), jnp.float32)],
    )(agg, agg, hr, g, b, wl, wr, bln)


def _final(agg, hr, g, b, batch3d):
    return pl.pallas_call(
        _final_body,
        grid=(NBLK,),
        in_specs=[_PART0, _PART1,
                  _row_spec((N, H)), _full_spec((1, H)), _full_spec((1, H)),
                  pl.BlockSpec((1, 1, BN), lambda i: (i, 0, 0))],
        out_specs=[_row_spec((N, H)), _full_spec((G, H))],
        out_shape=[jax.ShapeDtypeStruct((N, H), jnp.float32),
                   jax.ShapeDtypeStruct((G, H), jnp.float32)],
        scratch_shapes=[pltpu.VMEM((G, H), jnp.float32),
                        pltpu.VMEM((G, H), jnp.float32)],
    )(agg, agg, hr, g, b, batch3d)


def kernel(x, edge_index, batch, params):
    eflat = edge_index.reshape(2 * E)
    padv = jnp.arange(EPAD - E, dtype=jnp.int32) % CH
    batch3d = batch.reshape(NBLK, 1, BN)
    (wl1, bl1, wr1, g1, b1) = params[0]
    (wl2, bl2, wr2, g2, b2) = params[1]
    (wl3, bl3, wr3, g3, b3) = params[2]
    r = lambda v: v.reshape(1, H)
    spmm = _make_spmm()

    hlp1, hr1 = _proj0(x, wl1, wr1, r(bl1))
    agg1 = spmm(hlp1, eflat, padv, N + padv)
    hlp2, hr2 = _mid(agg1, hr1, r(g1), r(b1), wl2, wr2, r(bl2))
    agg2 = spmm(hlp2, eflat, padv, N + padv)
    hlp3, hr3 = _mid(agg2, hr2, r(g2), r(b2), wl3, wr3, r(bl3))
    agg3 = spmm(hlp3, eflat, padv, N + padv)
    node_emb, graph_emb = _final(agg3, hr3, r(g3), r(b3), batch3d)
    return (graph_emb, node_emb)
